# Initial kernel scaffold; baseline (speedup 1.0000x reference)
import jax
import jax.numpy as jnp
from jax.experimental import pallas as pl


def kernel(x, batch, W1a, b1a, W1b, b1b, W2a, b2a, W2b, b2b, W3a, b3a, W3b, b3b, Wl, bl, Wm1, bm1, Wm2, bm2, Wm3, bm3):
    raise NotImplementedError("write your pallas kernel here")


# speedup vs baseline: 2.7894x; 2.7894x over previous
import functools

import jax
import jax.numpy as jnp
from jax.experimental import pallas as pl

K = 30
_BIG = 3.0e38
_CD = (((1,), (1,)), ((), ()))
_CN = (((1,), (0,)), ((), ()))


def _mm(a, b, dims=_CN):
    return jax.lax.dot_general(a.astype(jnp.bfloat16),
                               b.astype(jnp.bfloat16), dims,
                               preferred_element_type=jnp.float32)


def _mm_exact(a, b, dims=_CN):
    bf16 = jnp.bfloat16
    f32 = jnp.float32
    b1 = b.astype(bf16)
    r1 = b - b1.astype(f32)
    b2 = r1.astype(bf16)
    b3 = (r1 - b2.astype(f32)).astype(bf16)
    ab = a.astype(bf16)
    out = jax.lax.dot_general(ab, b1, dims, preferred_element_type=f32)
    out = out + jax.lax.dot_general(ab, b2, dims, preferred_element_type=f32)
    out = out + jax.lax.dot_general(ab, b3, dims, preferred_element_type=f32)
    return out


def _edge_conv_body(xr_ref, xa_ref, brow_ref, bcol_ref, w1_ref, b1_ref,
                    w2_ref, b2_ref, o_ref):
    f32 = jnp.float32
    xr = xr_ref[...]
    xa = xa_ref[...]
    R = xr.shape[0]
    N = xa.shape[0]
    F = xr.shape[1]

    sqr = jnp.sum(xr * xr, axis=1, keepdims=True)
    xx = _mm(xr, xa, _CD)
    ones_row = jnp.ones((1, F), dtype=f32)
    sqa = _mm_exact(ones_row, xa * xa, _CD)
    d = (sqr - 2.0 * xx) + sqa

    brow = brow_ref[...]
    bcol = bcol_ref[...]
    d = jnp.where(brow != bcol, _BIG, d)

    iota = jax.lax.broadcasted_iota(jnp.int32, (R, N), 1)
    w1 = w1_ref[...]
    b1 = b1_ref[...]
    w2 = w2_ref[...]
    b2 = b2_ref[...]

    m_acc = jnp.full((R, 64), -_BIG, dtype=f32)
    for _ in range(K):
        mval = jnp.min(d, axis=1, keepdims=True)
        cand = jnp.where(d == mval, iota, N)
        idx = jnp.min(cand, axis=1, keepdims=True)
        onehot_b = iota == idx
        d = jnp.where(onehot_b, _BIG, d)
        onehot = onehot_b.astype(f32)
        xj = _mm_exact(onehot, xa)
        e = jnp.concatenate([xr, xj - xr], axis=1)
        h1 = jnp.maximum(_mm(e, w1) + b1, 0.0)
        h2 = jnp.maximum(_mm(h1, w2) + b2, 0.0)
        m_acc = jnp.maximum(m_acc, h2)
    o_ref[...] = m_acc


def _edge_conv(x, brow, bcol, W1, b1, W2, b2, block_rows):
    N, F = x.shape
    grid = (N // block_rows,)
    return pl.pallas_call(
        _edge_conv_body,
        grid=grid,
        in_specs=[
            pl.BlockSpec((block_rows, F), lambda i: (i, 0)),
            pl.BlockSpec((N, F), lambda i: (0, 0)),
            pl.BlockSpec((block_rows, 1), lambda i: (i, 0)),
            pl.BlockSpec((1, N), lambda i: (0, 0)),
            pl.BlockSpec(W1.shape, lambda i: (0, 0)),
            pl.BlockSpec((1, 64), lambda i: (0, 0)),
            pl.BlockSpec((64, 64), lambda i: (0, 0)),
            pl.BlockSpec((1, 64), lambda i: (0, 0)),
        ],
        out_specs=pl.BlockSpec((block_rows, 64), lambda i: (i, 0)),
        out_shape=jax.ShapeDtypeStruct((N, 64), jnp.float32),
    )(x, x, brow, bcol, W1, b1.reshape(1, 64), W2, b2.reshape(1, 64))


def _head_body(x1_ref, x2_ref, x3_ref, wl1_ref, wl2_ref, wl3_ref, bl_ref,
               wm1_ref, bm1_ref, wm2_ref, bm2_ref, wm3_ref, bm3_ref, o_ref):
    h = (_mm(x1_ref[...], wl1_ref[...])
         + _mm(x2_ref[...], wl2_ref[...])
         + _mm(x3_ref[...], wl3_ref[...])
         + bl_ref[...])
    h = jnp.maximum(h, 0.0)
    h = jnp.maximum(_mm(h, wm1_ref[...]) + bm1_ref[...], 0.0)
    h = jnp.maximum(_mm(h, wm2_ref[...]) + bm2_ref[...], 0.0)
    o_ref[...] = _mm(h, wm3_ref[...]) + bm3_ref[...]


def _head(x1, x2, x3, Wl, bl, Wm1, bm1, Wm2, bm2, Wm3, bm3, block_rows):
    N = x1.shape[0]
    grid = (N // block_rows,)
    row_spec = pl.BlockSpec((block_rows, 64), lambda i: (i, 0))

    def full(a):
        return pl.BlockSpec(a.shape, lambda i: (0, 0))

    Wl1, Wl2, Wl3 = Wl[:64], Wl[64:128], Wl[128:]
    bl2 = bl.reshape(1, -1)
    bm1_2 = bm1.reshape(1, -1)
    bm2_2 = bm2.reshape(1, -1)
    bm3_2 = bm3.reshape(1, -1)
    return pl.pallas_call(
        _head_body,
        grid=grid,
        in_specs=[
            row_spec, row_spec, row_spec,
            full(Wl1), full(Wl2), full(Wl3), full(bl2),
            full(Wm1), full(bm1_2), full(Wm2), full(bm2_2),
            full(Wm3), full(bm3_2),
        ],
        out_specs=pl.BlockSpec((block_rows, 64), lambda i: (i, 0)),
        out_shape=jax.ShapeDtypeStruct((N, 64), jnp.float32),
    )(x1, x2, x3, Wl1, Wl2, Wl3, bl2, Wm1, bm1_2, Wm2, bm2_2, Wm3, bm3_2)


@functools.partial(jax.jit, static_argnames=("block_rows",))
def _forward_impl(x, batch, W1a, b1a, W1b, b1b, W2a, b2a, W2b, b2b, W3a, b3a,
                  W3b, b3b, Wl, bl, Wm1, bm1, Wm2, bm2, Wm3, bm3,
                  block_rows=256):
    N = x.shape[0]
    bf = batch.astype(jnp.float32)
    brow = bf.reshape(N, 1)
    bcol = bf.reshape(1, N)
    x1 = _edge_conv(x, brow, bcol, W1a, b1a, W1b, b1b, block_rows)
    x2 = _edge_conv(x1, brow, bcol, W2a, b2a, W2b, b2b, block_rows)
    x3 = _edge_conv(x2, brow, bcol, W3a, b3a, W3b, b3b, block_rows)
    return _head(x1, x2, x3, Wl, bl, Wm1, bm1, Wm2, bm2, Wm3, bm3, block_rows)


def kernel(x, batch, W1a, b1a, W1b, b1b, W2a, b2a, W2b, b2b, W3a, b3a, W3b,
           b3b, Wl, bl, Wm1, bm1, Wm2, bm2, Wm3, bm3):
    return _forward_impl(x, batch, W1a, b1a, W1b, b1b, W2a, b2a, W2b, b2b,
                         W3a, b3a, W3b, b3b, Wl, bl, Wm1, bm1, Wm2, bm2,
                         Wm3, bm3)


# speedup vs baseline: 2.7897x; 1.0001x over previous
import functools

import jax
import jax.numpy as jnp
from jax.experimental import pallas as pl

K = 30
_BIG = 3.0e38
_CD = (((1,), (1,)), ((), ()))
_CN = (((1,), (0,)), ((), ()))


def _mm(a, b, dims=_CN):
    return jax.lax.dot_general(a.astype(jnp.bfloat16),
                               b.astype(jnp.bfloat16), dims,
                               preferred_element_type=jnp.float32)


def _split3(b):
    bf16 = jnp.bfloat16
    f32 = jnp.float32
    b1 = b.astype(bf16)
    r1 = b - b1.astype(f32)
    b2 = r1.astype(bf16)
    b3 = (r1 - b2.astype(f32)).astype(bf16)
    return b1, b2, b3


def _mm_parts(a, parts, dims=_CN):
    f32 = jnp.float32
    ab = a.astype(jnp.bfloat16)
    out = jax.lax.dot_general(ab, parts[0], dims, preferred_element_type=f32)
    out = out + jax.lax.dot_general(ab, parts[1], dims,
                                    preferred_element_type=f32)
    out = out + jax.lax.dot_general(ab, parts[2], dims,
                                    preferred_element_type=f32)
    return out


def _mm_exact(a, b, dims=_CN):
    return _mm_parts(a, _split3(b), dims)


def _edge_conv_body(xr_ref, xa_ref, brow_ref, bcol_ref, w1_ref, b1_ref,
                    w2_ref, b2_ref, o_ref):
    f32 = jnp.float32
    xr = xr_ref[...]
    xa = xa_ref[...]
    R = xr.shape[0]
    N = xa.shape[0]
    F = xr.shape[1]

    sqr = jnp.sum(xr * xr, axis=1, keepdims=True)
    xx = _mm(xr, xa, _CD)
    ones_row = jnp.ones((1, F), dtype=f32)
    sqa = _mm_exact(ones_row, xa * xa, _CD)
    d = (sqr - 2.0 * xx) + sqa

    brow = brow_ref[...]
    bcol = bcol_ref[...]
    d = jnp.where(brow != bcol, _BIG, d)

    iota = jax.lax.broadcasted_iota(jnp.int32, (R, N), 1)
    w1 = w1_ref[...]
    b1 = b1_ref[...]
    w2 = w2_ref[...]
    b2 = b2_ref[...]
    xa_parts = _split3(xa)

    m_acc = jnp.full((R, 64), -_BIG, dtype=f32)
    for _ in range(K):
        mval = jnp.min(d, axis=1, keepdims=True)
        cand = jnp.where(d == mval, iota, N)
        idx = jnp.min(cand, axis=1, keepdims=True)
        onehot_b = iota == idx
        d = jnp.where(onehot_b, _BIG, d)
        onehot = onehot_b.astype(f32)
        xj = _mm_parts(onehot, xa_parts)
        e = jnp.concatenate([xr, xj - xr], axis=1)
        h1 = jnp.maximum(_mm(e, w1) + b1, 0.0)
        h2 = jnp.maximum(_mm(h1, w2) + b2, 0.0)
        m_acc = jnp.maximum(m_acc, h2)
    o_ref[...] = m_acc


def _edge_conv(x, brow, bcol, W1, b1, W2, b2, block_rows):
    N, F = x.shape
    grid = (N // block_rows,)
    return pl.pallas_call(
        _edge_conv_body,
        grid=grid,
        in_specs=[
            pl.BlockSpec((block_rows, F), lambda i: (i, 0)),
            pl.BlockSpec((N, F), lambda i: (0, 0)),
            pl.BlockSpec((block_rows, 1), lambda i: (i, 0)),
            pl.BlockSpec((1, N), lambda i: (0, 0)),
            pl.BlockSpec(W1.shape, lambda i: (0, 0)),
            pl.BlockSpec((1, 64), lambda i: (0, 0)),
            pl.BlockSpec((64, 64), lambda i: (0, 0)),
            pl.BlockSpec((1, 64), lambda i: (0, 0)),
        ],
        out_specs=pl.BlockSpec((block_rows, 64), lambda i: (i, 0)),
        out_shape=jax.ShapeDtypeStruct((N, 64), jnp.float32),
    )(x, x, brow, bcol, W1, b1.reshape(1, 64), W2, b2.reshape(1, 64))


def _head_body(x1_ref, x2_ref, x3_ref, wl1_ref, wl2_ref, wl3_ref, bl_ref,
               wm1_ref, bm1_ref, wm2_ref, bm2_ref, wm3_ref, bm3_ref, o_ref):
    h = (_mm(x1_ref[...], wl1_ref[...])
         + _mm(x2_ref[...], wl2_ref[...])
         + _mm(x3_ref[...], wl3_ref[...])
         + bl_ref[...])
    h = jnp.maximum(h, 0.0)
    h = jnp.maximum(_mm(h, wm1_ref[...]) + bm1_ref[...], 0.0)
    h = jnp.maximum(_mm(h, wm2_ref[...]) + bm2_ref[...], 0.0)
    o_ref[...] = _mm(h, wm3_ref[...]) + bm3_ref[...]


def _head(x1, x2, x3, Wl, bl, Wm1, bm1, Wm2, bm2, Wm3, bm3, block_rows):
    N = x1.shape[0]
    grid = (N // block_rows,)
    row_spec = pl.BlockSpec((block_rows, 64), lambda i: (i, 0))

    def full(a):
        return pl.BlockSpec(a.shape, lambda i: (0, 0))

    Wl1, Wl2, Wl3 = Wl[:64], Wl[64:128], Wl[128:]
    bl2 = bl.reshape(1, -1)
    bm1_2 = bm1.reshape(1, -1)
    bm2_2 = bm2.reshape(1, -1)
    bm3_2 = bm3.reshape(1, -1)
    return pl.pallas_call(
        _head_body,
        grid=grid,
        in_specs=[
            row_spec, row_spec, row_spec,
            full(Wl1), full(Wl2), full(Wl3), full(bl2),
            full(Wm1), full(bm1_2), full(Wm2), full(bm2_2),
            full(Wm3), full(bm3_2),
        ],
        out_specs=pl.BlockSpec((block_rows, 64), lambda i: (i, 0)),
        out_shape=jax.ShapeDtypeStruct((N, 64), jnp.float32),
    )(x1, x2, x3, Wl1, Wl2, Wl3, bl2, Wm1, bm1_2, Wm2, bm2_2, Wm3, bm3_2)


@functools.partial(jax.jit, static_argnames=("block_rows",))
def _forward_impl(x, batch, W1a, b1a, W1b, b1b, W2a, b2a, W2b, b2b, W3a, b3a,
                  W3b, b3b, Wl, bl, Wm1, bm1, Wm2, bm2, Wm3, bm3,
                  block_rows=256):
    N = x.shape[0]
    bf = batch.astype(jnp.float32)
    brow = bf.reshape(N, 1)
    bcol = bf.reshape(1, N)
    x1 = _edge_conv(x, brow, bcol, W1a, b1a, W1b, b1b, block_rows)
    x2 = _edge_conv(x1, brow, bcol, W2a, b2a, W2b, b2b, block_rows)
    x3 = _edge_conv(x2, brow, bcol, W3a, b3a, W3b, b3b, block_rows)
    return _head(x1, x2, x3, Wl, bl, Wm1, bm1, Wm2, bm2, Wm3, bm3, block_rows)


def kernel(x, batch, W1a, b1a, W1b, b1b, W2a, b2a, W2b, b2b, W3a, b3a, W3b,
           b3b, Wl, bl, Wm1, bm1, Wm2, bm2, Wm3, bm3):
    return _forward_impl(x, batch, W1a, b1a, W1b, b1b, W2a, b2a, W2b, b2b,
                         W3a, b3a, W3b, b3b, Wl, bl, Wm1, bm1, Wm2, bm2,
                         Wm3, bm3)
